# baseline (speedup 1.0000x reference)
import functools

import jax
import jax.numpy as jnp
from jax.experimental import pallas as pl

_B, _C, _T, _K = 16, 512, 512, 4096
_N = _B * _T
_R = 256
_STEPS = _N // _R


def _vq_kernel(f_ref, t_ref, cb_ref, loss_ref, acc_ref):
    i = pl.program_id(0)
    f = f_ref[...]
    cb = cb_ref[...]

    dot = jax.lax.dot_general(
        f, cb, (((1,), (1,)), ((), ())),
        preferred_element_type=jnp.float32)

    x2 = jnp.sum(f * f, axis=1, keepdims=True)
    ones = jnp.ones((1, _C), dtype=jnp.float32)
    y2 = jax.lax.dot_general(
        ones, cb * cb, (((1,), (1,)), ((), ())),
        preferred_element_type=jnp.float32)

    d2 = jnp.maximum(x2 + y2 - 2.0 * dot, 0.0)

    t_row = t_ref[0]
    ir = jax.lax.broadcasted_iota(jnp.int32, (_R, _R), 0)
    ic = jax.lax.broadcasted_iota(jnp.int32, (_R, _R), 1)
    t_b = jnp.broadcast_to(t_row, (_R, _R))
    t_col = jnp.sum(jnp.where(ir == ic, t_b, 0), axis=1, keepdims=True)

    kiota = jax.lax.broadcasted_iota(jnp.int32, (_R, _K), 1)
    mask_t = kiota == t_col

    dot_t = jnp.sum(jnp.where(mask_t, dot, 0.0), axis=1, keepdims=True)
    y2_t = jnp.sum(jnp.where(mask_t, jnp.broadcast_to(y2, (_R, _K)), 0.0),
                   axis=1, keepdims=True)
    block_loss = jnp.sum(x2 + y2_t - 2.0 * dot_t).reshape(1, 1)

    dmin = jnp.min(d2, axis=1, keepdims=True)
    idx = jnp.min(jnp.where(d2 == dmin, kiota, _K), axis=1, keepdims=True)
    block_match = jnp.sum((idx == t_col).astype(jnp.float32)).reshape(1, 1)

    @pl.when(i == 0)
    def _init():
        loss_ref[...] = jnp.zeros((1, 1), jnp.float32)
        acc_ref[...] = jnp.zeros((1, 1), jnp.float32)

    loss_ref[...] += block_loss
    acc_ref[...] += block_match

    @pl.when(i == _STEPS - 1)
    def _final():
        loss_ref[...] = loss_ref[...] / float(_N * _C)
        acc_ref[...] = acc_ref[...] / float(_N)


@functools.partial(jax.jit, static_argnames=())
def kernel(student_features, teacher_codes, codebook, distance_matrix):
    del distance_matrix
    features = jnp.transpose(student_features, (0, 2, 1)).reshape(_N, _C)
    teacher = teacher_codes[0].reshape(_STEPS, 1, _R).astype(jnp.int32)

    loss, acc = pl.pallas_call(
        _vq_kernel,
        grid=(_STEPS,),
        in_specs=[
            pl.BlockSpec((_R, _C), lambda i: (i, 0)),
            pl.BlockSpec((1, 1, _R), lambda i: (i, 0, 0)),
            pl.BlockSpec((_K, _C), lambda i: (0, 0)),
        ],
        out_specs=[
            pl.BlockSpec((1, 1), lambda i: (0, 0)),
            pl.BlockSpec((1, 1), lambda i: (0, 0)),
        ],
        out_shape=[
            jax.ShapeDtypeStruct((1, 1), jnp.float32),
            jax.ShapeDtypeStruct((1, 1), jnp.float32),
        ],
    )(features, teacher, codebook)

    return (loss[0, 0], acc[0, 0])


# speedup vs baseline: 2.4871x; 2.4871x over previous
import functools

import jax
import jax.numpy as jnp
from jax.experimental import pallas as pl

_B, _C, _T, _K = 16, 512, 512, 4096
_N = _B * _T
_R = 256
_TB = _T // _R
_STEPS = _N // _R


def _vq_kernel(f_ref, t_ref, cb_ref, loss_ref, acc_ref):
    f = f_ref[0]
    cb = cb_ref[...]

    dot = jax.lax.dot_general(
        cb, f, (((1,), (0,)), ((), ())),
        preferred_element_type=jnp.float32)

    y2 = jnp.sum(cb * cb, axis=1, keepdims=True)
    score = y2 - 2.0 * dot

    t_row = t_ref[0]
    kio = jax.lax.broadcasted_iota(jnp.int32, (_K, _R), 0)
    mask_t = kio == t_row
    score_t = jnp.sum(jnp.where(mask_t, score, 0.0),
                      axis=0, keepdims=True)
    smin = jnp.min(score, axis=0, keepdims=True)

    x2 = jnp.sum(f * f, axis=0, keepdims=True)
    loss_ref[...] = jnp.sum(x2 + score_t).reshape(1, 1, 1)
    acc_ref[...] = jnp.sum(
        (score_t <= smin).astype(jnp.float32)).reshape(1, 1, 1)


@functools.partial(jax.jit, static_argnames=())
def kernel(student_features, teacher_codes, codebook, distance_matrix):
    del distance_matrix
    teacher = teacher_codes.reshape(_B, 1, _T).astype(jnp.int32)

    loss_p, acc_p = pl.pallas_call(
        _vq_kernel,
        grid=(_B, _TB),
        in_specs=[
            pl.BlockSpec((1, _C, _R), lambda b, tb: (b, 0, tb)),
            pl.BlockSpec((1, 1, _R), lambda b, tb: (b, 0, tb)),
            pl.BlockSpec((_K, _C), lambda b, tb: (0, 0)),
        ],
        out_specs=[
            pl.BlockSpec((1, 1, 1), lambda b, tb: (b * _TB + tb, 0, 0)),
            pl.BlockSpec((1, 1, 1), lambda b, tb: (b * _TB + tb, 0, 0)),
        ],
        out_shape=[
            jax.ShapeDtypeStruct((_STEPS, 1, 1), jnp.float32),
            jax.ShapeDtypeStruct((_STEPS, 1, 1), jnp.float32),
        ],
    )(student_features, teacher, codebook)

    loss = jnp.sum(loss_p) / float(_N * _C)
    accuracy = jnp.sum(acc_p) / float(_N)
    return (loss, accuracy)
